# baseline (speedup 1.0000x reference)
import functools

import jax
import jax.numpy as jnp
from jax import lax
from jax.experimental import pallas as pl
from jax.experimental.pallas import tpu as pltpu
from jax.experimental.pallas import tpu_sc as plsc

NC = 2
NS = 16
NW = NC * NS

B = 4096 * 200
D = 64
BPW = B // NW
C = 128
NCHUNK = BPW // C
NBUF = 4


def _mesh():
    return plsc.VectorSubcoreMesh(core_axis_name="c", subcore_axis_name="s")


@functools.partial(
    pl.kernel,
    out_type=jax.ShapeDtypeStruct((B, D), jnp.float32),
    mesh=_mesh(),
    scratch_types=[
        pltpu.VMEM((NCHUNK, C), jnp.int32),
        *[pltpu.VMEM((C, D), jnp.float32) for _ in range(NBUF)],
        *[pltpu.SemaphoreType.DMA for _ in range(NBUF)],
    ],
    compiler_params=pltpu.CompilerParams(use_tc_tiling_on_sc=False),
)
def _embed_lookup(idx_hbm, table_hbm, out_hbm, idx_v, *bufs_sems):
    rows = bufs_sems[:NBUF]
    sems = bufs_sems[NBUF:]
    wid = lax.axis_index("s") * NC + lax.axis_index("c")
    base = wid * BPW

    pltpu.sync_copy(idx_hbm.at[wid], idx_v)

    for b in range(NBUF):
        pltpu.async_copy(table_hbm.at[idx_v.at[b]], rows[b], sems[b])

    @pl.loop(0, NCHUNK, step=NBUF)
    def _(j):
        for b in range(NBUF):
            jj = j + b
            pltpu.make_async_copy(
                table_hbm.at[idx_v.at[jj]], rows[b], sems[b]
            ).wait()
            pltpu.sync_copy(rows[b], out_hbm.at[pl.ds(base + jj * C, C)])
            nxt = jj + NBUF

            @pl.when(nxt < NCHUNK)
            def _():
                pltpu.async_copy(table_hbm.at[idx_v.at[nxt]], rows[b], sems[b])


def kernel(x, embed_word):
    idx = x.reshape(NW, NCHUNK, C).astype(jnp.int32)
    out = _embed_lookup(idx, embed_word)
    return out.reshape(*x.shape, D)
